# Initial kernel scaffold; baseline (speedup 1.0000x reference)
import functools

import jax
import jax.numpy as jnp
from jax import lax
from jax.experimental import pallas as pl
from jax.experimental.pallas import tpu as pltpu
from jax.experimental.pallas import tpu_sc as plsc

VOCAB = 100000
DIM = 128
B = 4096 * 50
NC = 2
NS = 16
NW = NC * NS
B_PER_W = B // NW
CHUNK = 128
NCHUNK = B_PER_W // CHUNK


def _emb_body(idx_hbm, table_hbm, out_hbm, idx_v, rows_v, sem):
  wid = lax.axis_index("s") * NC + lax.axis_index("c")
  pltpu.sync_copy(idx_hbm.at[pl.ds(wid * NCHUNK, NCHUNK)], idx_v)

  def chunk(j, _):
    pltpu.async_copy(table_hbm.at[idx_v.at[j]], rows_v, sem).wait()
    pltpu.sync_copy(rows_v, out_hbm.at[pl.ds(wid * B_PER_W + j * CHUNK, CHUNK)])
    return 0

  lax.fori_loop(0, NCHUNK, chunk, 0)


@jax.jit
def _embed(idx2d, table):
  mesh = plsc.VectorSubcoreMesh(core_axis_name="c", subcore_axis_name="s")
  k = functools.partial(
      pl.kernel,
      mesh=mesh,
      out_type=jax.ShapeDtypeStruct((B, DIM), jnp.float32),
      scratch_types=[
          pltpu.VMEM((NCHUNK, CHUNK), jnp.int32),
          pltpu.VMEM((CHUNK, DIM), jnp.float32),
          pltpu.SemaphoreType.DMA,
      ],
  )(_emb_body)
  return k(idx2d, table)


def kernel(x, embed_weight):
  idx = x.astype(jnp.int32).reshape(NW * NCHUNK, CHUNK)
  out = _embed(idx, embed_weight)
  return out.reshape(x.shape[0], x.shape[1], DIM)


# speedup vs baseline: 2.9710x; 2.9710x over previous
import functools

import jax
import jax.numpy as jnp
from jax import lax
from jax.experimental import pallas as pl
from jax.experimental.pallas import tpu as pltpu
from jax.experimental.pallas import tpu_sc as plsc

VOCAB = 100000
DIM = 128
B = 4096 * 50
NC = 2
NS = 16
NW = NC * NS
B_PER_W = B // NW
CHUNK = 128
NCHUNK = B_PER_W // CHUNK


def _emb_body(idx_hbm, table_hbm, out_hbm, idx_v, rows_v, sem):
  wid = lax.axis_index("s") * NC + lax.axis_index("c")
  pltpu.sync_copy(idx_hbm.at[pl.ds(wid * B_PER_W, B_PER_W)], idx_v)

  def chunk(j, _):
    pltpu.async_copy(
        table_hbm.at[idx_v.at[pl.ds(j * CHUNK, CHUNK)]], rows_v, sem).wait()
    pltpu.sync_copy(rows_v, out_hbm.at[pl.ds(wid * B_PER_W + j * CHUNK, CHUNK)])
    return 0

  lax.fori_loop(0, NCHUNK, chunk, 0)


@jax.jit
def _embed(idx2d, table):
  mesh = plsc.VectorSubcoreMesh(core_axis_name="c", subcore_axis_name="s")
  k = functools.partial(
      pl.kernel,
      mesh=mesh,
      out_type=jax.ShapeDtypeStruct((B, DIM), jnp.float32),
      scratch_types=[
          pltpu.VMEM((B_PER_W,), jnp.int32),
          pltpu.VMEM((CHUNK, DIM), jnp.float32),
          pltpu.SemaphoreType.DMA,
      ],
  )(_emb_body)
  return k(idx2d, table)


def kernel(x, embed_weight):
  idx = x.astype(jnp.int32).reshape(B)
  out = _embed(idx, embed_weight)
  return out.reshape(x.shape[0], x.shape[1], DIM)


# speedup vs baseline: 3.3328x; 1.1218x over previous
import functools

import jax
import jax.numpy as jnp
from jax import lax
from jax.experimental import pallas as pl
from jax.experimental.pallas import tpu as pltpu
from jax.experimental.pallas import tpu_sc as plsc

VOCAB = 100000
DIM = 128
B = 4096 * 50
NC = 2
NS = 16
NW = NC * NS
B_PER_W = B // NW
CHUNK = 128
NCHUNK = B_PER_W // CHUNK
NBUF = 4
PF = 2


def _emb_body(idx_hbm, table_hbm, out_hbm, idx_v,
              b0, b1, b2, b3, g0, g1, g2, g3, w0, w1, w2, w3):
  bufs = [b0, b1, b2, b3]
  gsem = [g0, g1, g2, g3]
  wsem = [w0, w1, w2, w3]
  wid = lax.axis_index("s") * NC + lax.axis_index("c")
  base = wid * B_PER_W
  pltpu.sync_copy(idx_hbm.at[pl.ds(base, B_PER_W)], idx_v)

  def start_gather(j, b):
    pltpu.async_copy(
        table_hbm.at[idx_v.at[pl.ds(j * CHUNK, CHUNK)]], bufs[b], gsem[b])

  def wait_gather(j, b):
    pltpu.make_async_copy(
        table_hbm.at[idx_v.at[pl.ds(j * CHUNK, CHUNK)]], bufs[b],
        gsem[b]).wait()

  def start_write(j, b):
    pltpu.async_copy(
        bufs[b], out_hbm.at[pl.ds(base + j * CHUNK, CHUNK)], wsem[b])

  def wait_write(j, b):
    pltpu.make_async_copy(
        bufs[b], out_hbm.at[pl.ds(base + j * CHUNK, CHUNK)], wsem[b]).wait()

  start_gather(0, 0)
  start_gather(1, 1)
  start_gather(2, 2)
  wait_gather(0, 0)
  start_write(0, 0)
  start_gather(3, 3)
  wait_gather(1, 1)
  start_write(1, 1)
  wait_write(0, 0)
  start_gather(4, 0)
  wait_gather(2, 2)
  start_write(2, 2)
  wait_write(1, 1)
  start_gather(5, 1)
  wait_gather(3, 3)
  start_write(3, 3)

  def group(i, _):
    g = i * NBUF
    for b in range(NBUF):
      j = g + b
      bf = (b + PF) % NBUF
      wait_write(j - PF, bf)
      start_gather(j + PF, bf)
      wait_gather(j, b)
      start_write(j, b)
    return 0

  lax.fori_loop(1, NCHUNK // NBUF, group, 0)

  wait_gather(48, 0)
  start_write(48, 0)
  wait_gather(49, 1)
  start_write(49, 1)
  wait_write(46, 2)
  wait_write(47, 3)
  wait_write(48, 0)
  wait_write(49, 1)


@jax.jit
def _embed(idx1d, table):
  mesh = plsc.VectorSubcoreMesh(core_axis_name="c", subcore_axis_name="s")
  k = functools.partial(
      pl.kernel,
      mesh=mesh,
      out_type=jax.ShapeDtypeStruct((B, DIM), jnp.float32),
      scratch_types=(
          [pltpu.VMEM((B_PER_W,), jnp.int32)]
          + [pltpu.VMEM((CHUNK, DIM), jnp.float32)] * NBUF
          + [pltpu.SemaphoreType.DMA] * (2 * NBUF)
      ),
  )(_emb_body)
  return k(idx1d, table)


def kernel(x, embed_weight):
  idx = x.astype(jnp.int32).reshape(B)
  out = _embed(idx, embed_weight)
  return out.reshape(x.shape[0], x.shape[1], DIM)
